# Initial kernel scaffold; baseline (speedup 1.0000x reference)
import jax
import jax.numpy as jnp
from jax.experimental import pallas as pl


def kernel(input_ids, token_type_ids, word_embeddings, position_embeddings, token_type_embeddings, ln_gamma, ln_beta):
    raise NotImplementedError("write your pallas kernel here")


# speedup vs baseline: 2.0178x; 2.0178x over previous
import functools

import jax
import jax.numpy as jnp
from jax import lax
from jax.experimental import pallas as pl
from jax.experimental.pallas import tpu as pltpu
from jax.experimental.pallas import tpu_sc as plsc

B, S, V, H, P, T = 64, 512, 30522, 768, 512, 2
LN_EPS = 1e-12

N = B * S
NC, NS = 2, 16
NW = NC * NS
PER_W = N // NW
K = 64
NCHUNK = PER_W // K


def _sc_gather_body(ids_hbm, table_hbm, out_hbm, idx_v, buf0, buf1, sem):
    wid = lax.axis_index("s") * NC + lax.axis_index("c")
    base = wid * PER_W
    pltpu.sync_copy(ids_hbm.at[wid], idx_v)
    bufs = (buf0, buf1)
    cp = pltpu.async_copy(table_hbm.at[idx_v.at[0]], buf0, sem)
    for c in range(NCHUNK):
        cur = bufs[c % 2]
        nxt = bufs[(c + 1) % 2]
        cp.wait()
        if c + 1 < NCHUNK:
            cp = pltpu.async_copy(table_hbm.at[idx_v.at[c + 1]], nxt, sem)
        pltpu.sync_copy(cur, out_hbm.at[pl.ds(base + c * K, K)])


_sc_gather = functools.partial(
    pl.kernel,
    mesh=plsc.VectorSubcoreMesh(core_axis_name="c", subcore_axis_name="s"),
    out_type=jax.ShapeDtypeStruct((N, H), jnp.float32),
    scratch_types=[
        pltpu.VMEM((NCHUNK, K), jnp.int32),
        pltpu.VMEM((K, H), jnp.float32),
        pltpu.VMEM((K, H), jnp.float32),
        pltpu.SemaphoreType.DMA,
    ],
)(_sc_gather_body)


def _tc_ln_body(words_ref, tt_ref, pos_ref, type_ref, gamma_ref, beta_ref,
                out_ref):
    x = words_ref[0]
    tt = tt_ref[0, 0]
    pos = pos_ref[...]
    t0 = type_ref[0:1, :]
    t1 = type_ref[1:2, :]
    ttf = tt.astype(jnp.float32)[:, None]
    emb = x + pos + t0 + ttf * (t1 - t0)
    mu = jnp.mean(emb, axis=-1, keepdims=True)
    d = emb - mu
    var = jnp.mean(d * d, axis=-1, keepdims=True)
    rstd = lax.rsqrt(var + LN_EPS)
    out_ref[0] = (d * rstd) * gamma_ref[...] + beta_ref[...]


def _tc_ln(words, token_type_ids, pos_emb, type_emb, gamma, beta):
    grid = (B,)
    return pl.pallas_call(
        _tc_ln_body,
        grid=grid,
        in_specs=[
            pl.BlockSpec((1, S, H), lambda b: (b, 0, 0)),
            pl.BlockSpec((1, 1, S), lambda b: (b, 0, 0)),
            pl.BlockSpec((S, H), lambda b: (0, 0)),
            pl.BlockSpec((T, H), lambda b: (0, 0)),
            pl.BlockSpec((1, H), lambda b: (0, 0)),
            pl.BlockSpec((1, H), lambda b: (0, 0)),
        ],
        out_specs=pl.BlockSpec((1, S, H), lambda b: (b, 0, 0)),
        out_shape=jax.ShapeDtypeStruct((B, S, H), jnp.float32),
    )(words, token_type_ids, pos_emb, type_emb, gamma, beta)


def kernel(input_ids, token_type_ids, word_embeddings, position_embeddings,
           token_type_embeddings, ln_gamma, ln_beta):
    ids = input_ids.reshape(NW, NCHUNK, K)
    words = _sc_gather(ids, word_embeddings)
    words = words.reshape(B, S, H)
    out = _tc_ln(
        words,
        token_type_ids.reshape(B, 1, S),
        position_embeddings,
        token_type_embeddings,
        ln_gamma.reshape(1, H),
        ln_beta.reshape(1, H),
    )
    return out


# speedup vs baseline: 2.3281x; 1.1538x over previous
import functools

import jax
import jax.numpy as jnp
from jax import lax
from jax.experimental import pallas as pl
from jax.experimental.pallas import tpu as pltpu
from jax.experimental.pallas import tpu_sc as plsc

B, S, V, H, P, T = 64, 512, 30522, 768, 512, 2
LN_EPS = 1e-12

N = B * S
NC, NS = 2, 16
NW = NC * NS
PER_W = N // NW
K = 64
NCHUNK = PER_W // K


def _sc_gather_body(ids_hbm, table_hbm, out_hbm, idx_v, buf0, buf1, sem):
    wid = lax.axis_index("s") * NC + lax.axis_index("c")
    base = wid * PER_W
    pltpu.sync_copy(ids_hbm.at[wid], idx_v)
    bufs = (buf0, buf1)
    cp = pltpu.async_copy(table_hbm.at[idx_v.at[0]], buf0, sem)
    for c in range(NCHUNK):
        cur = bufs[c % 2]
        nxt = bufs[(c + 1) % 2]
        cp.wait()
        if c + 1 < NCHUNK:
            cp = pltpu.async_copy(table_hbm.at[idx_v.at[c + 1]], nxt, sem)
        pltpu.sync_copy(cur, out_hbm.at[pl.ds(base + c * K, K)])


_sc_gather = functools.partial(
    pl.kernel,
    mesh=plsc.VectorSubcoreMesh(core_axis_name="c", subcore_axis_name="s"),
    out_type=jax.ShapeDtypeStruct((N, H), jnp.float32),
    scratch_types=[
        pltpu.VMEM((NCHUNK, K), jnp.int32),
        pltpu.VMEM((K, H), jnp.float32),
        pltpu.VMEM((K, H), jnp.float32),
        pltpu.SemaphoreType.DMA,
    ],
)(_sc_gather_body)


BB = 4


def _tc_ln_body(words_ref, tt_ref, pos_ref, type_ref, gamma_ref, beta_ref,
                out_ref):
    x = words_ref[...]
    tt = tt_ref[:, 0, :]
    pos = pos_ref[...][None]
    t0 = type_ref[0:1, :][None]
    t1 = type_ref[1:2, :][None]
    ttf = tt.astype(jnp.float32)[:, :, None]
    emb = x + pos + t0 + ttf * (t1 - t0)
    mu = jnp.mean(emb, axis=-1, keepdims=True)
    d = emb - mu
    var = jnp.mean(d * d, axis=-1, keepdims=True)
    rstd = lax.rsqrt(var + LN_EPS)
    out_ref[...] = (d * rstd) * gamma_ref[...][None] + beta_ref[...][None]


def _tc_ln(words, token_type_ids, pos_emb, type_emb, gamma, beta):
    grid = (B // BB,)
    return pl.pallas_call(
        _tc_ln_body,
        grid=grid,
        in_specs=[
            pl.BlockSpec((BB, S, H), lambda b: (b, 0, 0)),
            pl.BlockSpec((BB, 1, S), lambda b: (b, 0, 0)),
            pl.BlockSpec((S, H), lambda b: (0, 0)),
            pl.BlockSpec((T, H), lambda b: (0, 0)),
            pl.BlockSpec((1, H), lambda b: (0, 0)),
            pl.BlockSpec((1, H), lambda b: (0, 0)),
        ],
        out_specs=pl.BlockSpec((BB, S, H), lambda b: (b, 0, 0)),
        out_shape=jax.ShapeDtypeStruct((B, S, H), jnp.float32),
    )(words, token_type_ids, pos_emb, type_emb, gamma, beta)


def kernel(input_ids, token_type_ids, word_embeddings, position_embeddings,
           token_type_embeddings, ln_gamma, ln_beta):
    ids = input_ids.reshape(NW, NCHUNK, K)
    words = _sc_gather(ids, word_embeddings)
    words = words.reshape(B, S, H)
    out = _tc_ln(
        words,
        token_type_ids.reshape(B, 1, S),
        position_embeddings,
        token_type_embeddings,
        ln_gamma.reshape(1, H),
        ln_beta.reshape(1, H),
    )
    return out
